# baseline (speedup 1.0000x reference)
import functools

import jax
import jax.numpy as jnp
from jax import lax
from jax.experimental import pallas as pl
from jax.experimental.pallas import tpu as pltpu
from jax.experimental.pallas import tpu_sc as plsc

N_TOTAL = 100000
D = 32
N_REFLN = 1000000
N_ASUS = 2
GRID = 121

NC = 2
NS = 16
NW = NC * NS

DPAD = 128
CHUNK = 320
NCHUNK = N_REFLN // CHUNK
M_MAX = (NCHUNK + NW - 1) // NW
N_IT = ((M_MAX + 3 + 1) // 2) * 2


def _sc_body(source_hbm, rasu_hbm, h_hbm, k_hbm, l_hbm, grid_hbm, out_hbm,
             rasu0, rasu1, hv0, hv1, kv0, kv1, lv0, lv1,
             flat0, flat1, idx0, idx1, rows_v, pack_v,
             sin0, sin1, sg0, sg1, sr0, sr1, so0, so1):
    cid = lax.axis_index("c")
    sid = lax.axis_index("s")
    wid = sid * NC + cid

    sin = (sin0, sin1)
    sg = (sg0, sg1)
    sr = (sr0, sr1)
    so = (so0, so1)
    rasu_s = (rasu0, rasu1)
    h_s = (hv0, hv1)
    k_s = (kv0, kv1)
    l_s = (lv0, lv1)
    flat_s = (flat0, flat1)
    idx_s = (idx0, idx1)

    def chunk_id(jj):
        return wid + jj * NW

    def ok(jj):
        return (jj >= 0) & (chunk_id(jj) < NCHUNK)

    def in_descs(s, jj):
        base = chunk_id(jj) * CHUNK
        return (
            pltpu.make_async_copy(rasu_hbm.at[pl.ds(base, CHUNK)],
                                  rasu_s[s], sin[s]),
            pltpu.make_async_copy(h_hbm.at[pl.ds(base, CHUNK)],
                                  h_s[s], sin[s]),
            pltpu.make_async_copy(k_hbm.at[pl.ds(base, CHUNK)],
                                  k_s[s], sin[s]),
            pltpu.make_async_copy(l_hbm.at[pl.ds(base, CHUNK)],
                                  l_s[s], sin[s]),
        )

    def grid_desc(s):
        return pltpu.make_async_copy(grid_hbm.at[flat_s[s]],
                                     idx_s[s], sg[s])

    def rows_desc(s):
        return pltpu.make_async_copy(source_hbm.at[idx_s[s]],
                                     rows_v.at[s], sr[s])

    def out_desc(s, jj):
        obase = pl.multiple_of(chunk_id(jj) * (CHUNK * D), 8)
        return pltpu.make_async_copy(
            pack_v.at[s], out_hbm.at[pl.ds(obase, CHUNK * D)], so[s])

    def body(j, b):
        s3 = (b + 1) % 2

        @pl.when(ok(j - 5))
        def _():
            out_desc(s3, j - 5).wait()

        @pl.when(ok(j - 3))
        def _():
            rows_desc(s3).wait()

            def pack_iter(p, carry):
                r4 = p * 4
                for jj in range(4):
                    for c in range(2):
                        v = rows_v[s3, r4 + jj, pl.ds(c * 16, 16)]
                        pack_v[s3, pl.ds((r4 + jj) * D + c * 16, 16)] = v
                return carry

            lax.fori_loop(0, CHUNK // 4, pack_iter, 0, unroll=2)
            out_desc(s3, j - 3).start()

        s2 = b

        @pl.when(ok(j - 2))
        def _():
            grid_desc(s2).wait()
            rows_desc(s2).start()

        s1 = (b + 1) % 2

        @pl.when(ok(j - 1))
        def _():
            for d in in_descs(s1, j - 1):
                d.wait()

            def idx_iter(i, carry):
                b16 = i * 16
                rasu = rasu_s[s1][pl.ds(b16, 16)]
                h = h_s[s1][pl.ds(b16, 16)]
                k = k_s[s1][pl.ds(b16, 16)]
                l = l_s[s1][pl.ds(b16, 16)]
                flat = ((rasu * GRID + h) * GRID + k) * GRID + l
                flat_s[s1][pl.ds(b16, 16)] = flat
                return carry

            lax.fori_loop(0, CHUNK // 16, idx_iter, 0, unroll=4)
            grid_desc(s1).start()

        s0 = b

        @pl.when(ok(j))
        def _():
            for d in in_descs(s0, j):
                d.start()

    def pair_iter(t, carry):
        for b in range(2):
            body(2 * t + b, b)
        return carry

    lax.fori_loop(0, N_IT // 2, pair_iter, 0)

    for e in range(2):
        j = N_IT + e
        s = (j + 1) % 2

        @pl.when(ok(j - 5))
        def _():
            out_desc(s, j - 5).wait()


@jax.jit
def _sc_gather(source_pad, rasu_id, h, k, l, grid_flat):
    mesh = plsc.VectorSubcoreMesh(core_axis_name="c", subcore_axis_name="s")
    kern = pl.kernel(
        _sc_body,
        out_type=jax.ShapeDtypeStruct((N_REFLN * D,), jnp.float32),
        mesh=mesh,
        scratch_types=[
        ] + [pltpu.VMEM((CHUNK,), jnp.int32)] * 12 + [
            pltpu.VMEM((2, CHUNK, DPAD), jnp.float32),
            pltpu.VMEM((2, CHUNK * D), jnp.float32),
        ] + [pltpu.SemaphoreType.DMA] * 8,
    )
    return kern(source_pad, rasu_id, h, k, l, grid_flat)


def kernel(source, rasu_id, H, reflection_id_grid):
    h = H[:, 0]
    k = H[:, 1]
    l = H[:, 2]
    grid_flat = reflection_id_grid.reshape(-1)
    source_pad = jnp.pad(source, ((0, 0), (0, DPAD - D)))
    out_flat = _sc_gather(source_pad, rasu_id, h, k, l, grid_flat)
    return out_flat.reshape(N_REFLN, D)


# speedup vs baseline: 1.1582x; 1.1582x over previous
import functools

import jax
import jax.numpy as jnp
from jax import lax
from jax.experimental import pallas as pl
from jax.experimental.pallas import tpu as pltpu
from jax.experimental.pallas import tpu_sc as plsc

N_TOTAL = 100000
D = 32
N_REFLN = 1000000
N_ASUS = 2
GRID = 121

NC = 2
NS = 16
NW = NC * NS

DPAD = 128
CHUNK = 320
NCHUNK = N_REFLN // CHUNK
M_MAX = (NCHUNK + NW - 1) // NW
N_IT = ((M_MAX + 3 + 1) // 2) * 2


def _sc_body(source_hbm, rasu_hbm, h_hbm, k_hbm, l_hbm, grid_hbm, out_hbm,
             rasu0, rasu1, hv0, hv1, kv0, kv1, lv0, lv1,
             flat0, flat1, idx0, idx1, rows_v, pack_v,
             sin0, sin1, sg0, sg1, sr0, sr1, so0, so1):
    cid = lax.axis_index("c")
    sid = lax.axis_index("s")
    wid = sid * NC + cid

    sin = (sin0, sin1)
    sg = (sg0, sg1)
    sr = (sr0, sr1)
    so = (so0, so1)
    rasu_s = (rasu0, rasu1)
    h_s = (hv0, hv1)
    k_s = (kv0, kv1)
    l_s = (lv0, lv1)
    flat_s = (flat0, flat1)
    idx_s = (idx0, idx1)

    def chunk_id(jj):
        return wid + jj * NW

    def ok(jj):
        return (jj >= 0) & (chunk_id(jj) < NCHUNK)

    def in_descs(s, jj):
        base = chunk_id(jj) * CHUNK
        return (
            pltpu.make_async_copy(rasu_hbm.at[pl.ds(base, CHUNK)],
                                  rasu_s[s], sin[s]),
            pltpu.make_async_copy(h_hbm.at[pl.ds(base, CHUNK)],
                                  h_s[s], sin[s]),
            pltpu.make_async_copy(k_hbm.at[pl.ds(base, CHUNK)],
                                  k_s[s], sin[s]),
            pltpu.make_async_copy(l_hbm.at[pl.ds(base, CHUNK)],
                                  l_s[s], sin[s]),
        )

    def grid_desc(s):
        return pltpu.make_async_copy(grid_hbm.at[flat_s[s]],
                                     idx_s[s], sg[s])

    def rows_desc(s):
        return pltpu.make_async_copy(source_hbm.at[idx_s[s]],
                                     rows_v.at[s], sr[s])

    def out_desc(s, jj):
        obase = pl.multiple_of(chunk_id(jj) * (CHUNK // 4), 8)
        return pltpu.make_async_copy(
            pack_v.at[s], out_hbm.at[pl.ds(obase, CHUNK // 4)], so[s])

    def body(j, b):
        s3 = (b + 1) % 2

        @pl.when(ok(j - 5))
        def _():
            out_desc(s3, j - 5).wait()

        @pl.when(ok(j - 3))
        def _():
            rows_desc(s3).wait()

            def pack_iter(p, carry):
                r4 = p * 4
                for jj in range(4):
                    for c in range(2):
                        v = rows_v[s3, r4 + jj, pl.ds(c * 16, 16)]
                        pack_v[s3, p, pl.ds(jj * D + c * 16, 16)] = v
                return carry

            lax.fori_loop(0, CHUNK // 4, pack_iter, 0, unroll=2)
            out_desc(s3, j - 3).start()

        s2 = b

        @pl.when(ok(j - 2))
        def _():
            grid_desc(s2).wait()
            rows_desc(s2).start()

        s1 = (b + 1) % 2

        @pl.when(ok(j - 1))
        def _():
            for d in in_descs(s1, j - 1):
                d.wait()

            def idx_iter(i, carry):
                b16 = i * 16
                rasu = rasu_s[s1][pl.ds(b16, 16)]
                h = h_s[s1][pl.ds(b16, 16)]
                k = k_s[s1][pl.ds(b16, 16)]
                l = l_s[s1][pl.ds(b16, 16)]
                flat = ((rasu * GRID + h) * GRID + k) * GRID + l
                flat_s[s1][pl.ds(b16, 16)] = flat
                return carry

            lax.fori_loop(0, CHUNK // 16, idx_iter, 0, unroll=4)
            grid_desc(s1).start()

        s0 = b

        @pl.when(ok(j))
        def _():
            for d in in_descs(s0, j):
                d.start()

    def pair_iter(t, carry):
        for b in range(2):
            body(2 * t + b, b)
        return carry

    lax.fori_loop(0, N_IT // 2, pair_iter, 0)

    for e in range(2):
        j = N_IT + e
        s = (j + 1) % 2

        @pl.when(ok(j - 5))
        def _():
            out_desc(s, j - 5).wait()


@jax.jit
def _sc_gather(source_pad, rasu_id, h, k, l, grid_flat):
    mesh = plsc.VectorSubcoreMesh(core_axis_name="c", subcore_axis_name="s")
    kern = pl.kernel(
        _sc_body,
        out_type=jax.ShapeDtypeStruct((N_REFLN // 4, DPAD), jnp.float32),
        mesh=mesh,
        scratch_types=[
        ] + [pltpu.VMEM((CHUNK,), jnp.int32)] * 12 + [
            pltpu.VMEM((2, CHUNK, DPAD), jnp.float32),
            pltpu.VMEM((2, CHUNK // 4, DPAD), jnp.float32),
        ] + [pltpu.SemaphoreType.DMA] * 8,
    )
    return kern(source_pad, rasu_id, h, k, l, grid_flat)


def kernel(source, rasu_id, H, reflection_id_grid):
    h = H[:, 0]
    k = H[:, 1]
    l = H[:, 2]
    grid_flat = reflection_id_grid.reshape(-1)
    source_pad = jnp.pad(source, ((0, 0), (0, DPAD - D)))
    out_pack = _sc_gather(source_pad, rasu_id, h, k, l, grid_flat)
    out_t = out_pack.reshape(N_REFLN // 4, 4, D).transpose(2, 0, 1)
    return out_t.reshape(D, N_REFLN).T


# speedup vs baseline: 1.2917x; 1.1153x over previous
import functools

import jax
import jax.numpy as jnp
from jax import lax
from jax.experimental import pallas as pl
from jax.experimental.pallas import tpu as pltpu
from jax.experimental.pallas import tpu_sc as plsc

N_TOTAL = 100000
D = 32
N_REFLN = 1000000
N_ASUS = 2
GRID = 121

NC = 2
NS = 16
NW = NC * NS

DPAD = 128
CHUNK = 320
SEG = CHUNK // 4
STRIDE = N_REFLN // 4
NCHUNK = N_REFLN // CHUNK
M_MAX = (NCHUNK + NW - 1) // NW
N_IT = ((M_MAX + 3 + 1) // 2) * 2


def _sc_body(source_hbm, rasu_hbm, h_hbm, k_hbm, l_hbm, grid_hbm, out_hbm,
             rasu0, rasu1, hv0, hv1, kv0, kv1, lv0, lv1,
             flat0, flat1, idx0, idx1, rows_v, pack_v,
             sin0, sin1, sg0, sg1, sr0, sr1, so0, so1):
    cid = lax.axis_index("c")
    sid = lax.axis_index("s")
    wid = sid * NC + cid

    sin = (sin0, sin1)
    sg = (sg0, sg1)
    sr = (sr0, sr1)
    so = (so0, so1)
    rasu_s = (rasu0, rasu1)
    h_s = (hv0, hv1)
    k_s = (kv0, kv1)
    l_s = (lv0, lv1)
    flat_s = (flat0, flat1)
    idx_s = (idx0, idx1)

    def chunk_id(jj):
        return wid + jj * NW

    def ok(jj):
        return (jj >= 0) & (chunk_id(jj) < NCHUNK)

    def in_descs(s, jj):
        descs = []
        for q in range(4):
            base = chunk_id(jj) * SEG + q * STRIDE
            for hbm, vbuf in ((rasu_hbm, rasu_s[s]), (h_hbm, h_s[s]),
                              (k_hbm, k_s[s]), (l_hbm, l_s[s])):
                descs.append(pltpu.make_async_copy(
                    hbm.at[pl.ds(base, SEG)],
                    vbuf.at[q, pl.ds(0, SEG)], sin[s]))
        return descs

    def grid_desc(s):
        return pltpu.make_async_copy(grid_hbm.at[flat_s[s]],
                                     idx_s[s], sg[s])

    def rows_desc(s):
        return pltpu.make_async_copy(source_hbm.at[idx_s[s]],
                                     rows_v.at[s], sr[s])

    def out_desc(s, jj):
        obase = pl.multiple_of(chunk_id(jj) * SEG, 8)
        return pltpu.make_async_copy(
            pack_v.at[s], out_hbm.at[pl.ds(obase, SEG)], so[s])

    def body(j, b):
        s3 = (b + 1) % 2

        @pl.when(ok(j - 5))
        def _():
            out_desc(s3, j - 5).wait()

        @pl.when(ok(j - 3))
        def _():
            rows_desc(s3).wait()

            def pack_iter(p, carry):
                for jj in range(4):
                    for c in range(2):
                        v = rows_v[s3, jj * SEG + p, pl.ds(c * 16, 16)]
                        pack_v[s3, p, pl.ds(jj * D + c * 16, 16)] = v
                return carry

            lax.fori_loop(0, SEG, pack_iter, 0, unroll=2)
            out_desc(s3, j - 3).start()

        s2 = b

        @pl.when(ok(j - 2))
        def _():
            grid_desc(s2).wait()
            rows_desc(s2).start()

        s1 = (b + 1) % 2

        @pl.when(ok(j - 1))
        def _():
            for d in in_descs(s1, j - 1):
                d.wait()

            for q in range(4):
                def idx_iter(i, carry, q=q):
                    b16 = i * 16
                    rasu = rasu_s[s1][q, pl.ds(b16, 16)]
                    h = h_s[s1][q, pl.ds(b16, 16)]
                    k = k_s[s1][q, pl.ds(b16, 16)]
                    l = l_s[s1][q, pl.ds(b16, 16)]
                    flat = ((rasu * GRID + h) * GRID + k) * GRID + l
                    flat_s[s1][pl.ds(q * SEG + b16, 16)] = flat
                    return carry

                lax.fori_loop(0, SEG // 16, idx_iter, 0, unroll=5)
            grid_desc(s1).start()

        s0 = b

        @pl.when(ok(j))
        def _():
            for d in in_descs(s0, j):
                d.start()

    def pair_iter(t, carry):
        for b in range(2):
            body(2 * t + b, b)
        return carry

    lax.fori_loop(0, N_IT // 2, pair_iter, 0)

    for e in range(2):
        j = N_IT + e
        s = (j + 1) % 2

        @pl.when(ok(j - 5))
        def _():
            out_desc(s, j - 5).wait()


@jax.jit
def _sc_gather(source_pad, rasu_id, h, k, l, grid_flat):
    mesh = plsc.VectorSubcoreMesh(core_axis_name="c", subcore_axis_name="s")
    kern = pl.kernel(
        _sc_body,
        out_type=jax.ShapeDtypeStruct((N_REFLN // 4, DPAD), jnp.float32),
        mesh=mesh,
        scratch_types=[
        ] + [pltpu.VMEM((4, DPAD), jnp.int32)] * 8
          + [pltpu.VMEM((CHUNK,), jnp.int32)] * 4 + [
            pltpu.VMEM((2, CHUNK, DPAD), jnp.float32),
            pltpu.VMEM((2, SEG, DPAD), jnp.float32),
        ] + [pltpu.SemaphoreType.DMA] * 8,
    )
    return kern(source_pad, rasu_id, h, k, l, grid_flat)


def kernel(source, rasu_id, H, reflection_id_grid):
    h = H[:, 0]
    k = H[:, 1]
    l = H[:, 2]
    grid_flat = reflection_id_grid.reshape(-1)
    source_pad = jnp.pad(source, ((0, 0), (0, DPAD - D)))
    out_pack = _sc_gather(source_pad, rasu_id, h, k, l, grid_flat)
    out_t = out_pack.reshape(STRIDE, 4, D).transpose(2, 1, 0)
    return out_t.reshape(D, N_REFLN).T
